# baseline (speedup 1.0000x reference)
import functools

import jax
import jax.numpy as jnp
from jax import lax
from jax.experimental import pallas as pl
from jax.experimental.pallas import tpu as pltpu
from jax.experimental.pallas import tpu_sc as plsc

N_SPLIT = 448
SC_GROUP = 8
SC_NBUF = 4


def _tc_body(ip_ref, h_ref, hs_ref, skip_ref, br_ref, ti_ref, fi_ref,
             out_ip_ref, out_h_ref, acc_h_ref, *, block_i: int):
    k = pl.program_id(0)
    nk = pl.num_programs(0)
    bi = block_i
    n = skip_ref.shape[1]

    @pl.when(k == 0)
    def _segment_part():
        acc_h_ref[...] = jnp.zeros_like(acc_h_ref)

        rows = jax.lax.broadcasted_iota(jnp.int32, (n, n), 0)
        cols = jax.lax.broadcasted_iota(jnp.int32, (n, n), 1)
        skip_full = skip_ref[...]
        no_skip = jnp.sum(jnp.where(rows == cols, skip_full, 0.0),
                          axis=1, keepdims=True)
        ip = ip_ref[...]
        pbt = ip * no_skip * br_ref[:, 0:1]
        pbf = ip * no_skip * br_ref[:, 1:2]

        ot = (ti_ref[...] == rows).astype(jnp.float32)
        of = (fi_ref[...] == rows).astype(jnp.float32)

        dn = (((1,), (0,)), ((), ()))
        hp = jax.lax.Precision.HIGHEST
        ip_t = jax.lax.dot_general(ot, pbt, dn, precision=hp,
                                   preferred_element_type=jnp.float32)
        ip_f = jax.lax.dot_general(of, pbf, dn, precision=hp,
                                   preferred_element_type=jnp.float32)

        nt = n - N_SPLIT
        t_rows = N_SPLIT + jax.lax.broadcasted_iota(jnp.int32, (nt, n), 0)
        t_cols = jax.lax.broadcasted_iota(jnp.int32, (nt, n), 1)
        w_tail = jnp.where(t_rows != t_cols,
                           skip_ref[pl.ds(N_SPLIT, nt), :], 0.0)
        w_tail = w_tail * ip_ref[pl.ds(N_SPLIT, nt), :]
        tail_ip = jax.lax.dot_general(
            w_tail, jnp.ones((nt, 1), jnp.float32), (((0,), (0,)), ((), ())),
            preferred_element_type=jnp.float32)

        out_ip_ref[...] = ip_t + ip_f + tail_ip

        h = h_ref[...]
        th = jax.lax.dot_general(ot, h * pbt, dn, precision=hp,
                                 preferred_element_type=jnp.float32)
        fh = jax.lax.dot_general(of, h * pbf, dn, precision=hp,
                                 preferred_element_type=jnp.float32)
        out_h_ref[...] = th + fh

    row_ids = k * bi + jax.lax.broadcasted_iota(jnp.int32, (bi, n), 0)
    col_ids = jax.lax.broadcasted_iota(jnp.int32, (bi, n), 1)
    skip_blk = skip_ref[pl.ds(k * bi, bi), :]
    ip_blk = ip_ref[pl.ds(k * bi, bi), :]

    w = jnp.where(row_ids != col_ids, skip_blk, 0.0) * ip_blk

    ones = jnp.ones((bi, 1), jnp.float32)
    out_ip_ref[...] += jax.lax.dot_general(
        w, ones, (((0,), (0,)), ((), ())),
        preferred_element_type=jnp.float32)

    cj = 64
    for jc in range(n // cj):
        js = jc * cj
        hs_c = hs_ref[:, js:js + cj, :]
        w_c = w[:, js:js + cj]
        acc_h_ref[js:js + cj, :] += jnp.sum(hs_c * w_c[:, :, None], axis=0)

    @pl.when(k == nk - 1)
    def _finish():
        out_h_ref[...] += acc_h_ref[...]


def _tc_partial(ip2, h, hs, skip, br, ti2, fi2):
    n = skip.shape[0]
    h_dim = h.shape[1]
    block_i = 64
    nk = N_SPLIT // block_i
    return pl.pallas_call(
        functools.partial(_tc_body, block_i=block_i),
        grid=(nk,),
        in_specs=[
            pl.BlockSpec((n, 1), lambda k: (0, 0)),
            pl.BlockSpec((n, h_dim), lambda k: (0, 0)),
            pl.BlockSpec((block_i, n, h_dim), lambda k: (k, 0, 0)),
            pl.BlockSpec((n, n), lambda k: (0, 0)),
            pl.BlockSpec((n, 2), lambda k: (0, 0)),
            pl.BlockSpec((1, n), lambda k: (0, 0)),
            pl.BlockSpec((1, n), lambda k: (0, 0)),
        ],
        out_specs=[
            pl.BlockSpec((n, 1), lambda k: (0, 0)),
            pl.BlockSpec((n, h_dim), lambda k: (0, 0)),
        ],
        out_shape=[
            jax.ShapeDtypeStruct((n, 1), jnp.float32),
            jax.ShapeDtypeStruct((n, h_dim), jnp.float32),
        ],
        scratch_shapes=[
            pltpu.VMEM((n, h_dim), jnp.float32),
        ],
    )(ip2, h, hs, skip, br, ti2, fi2)


def _make_sc_partial(n, h_dim):
    info = plsc.get_sparse_core_info()
    nc, ns, nl = info.num_cores, info.num_subcores, info.num_lanes
    nw = nc * ns
    jpw = n // nw
    g = SC_GROUP
    n_rows = n - N_SPLIT
    ngrp = n_rows // g
    nh = h_dim // nl

    mesh = plsc.VectorSubcoreMesh(core_axis_name="c", subcore_axis_name="s")

    @functools.partial(
        pl.kernel, mesh=mesh,
        out_type=jax.ShapeDtypeStruct((n, h_dim), jnp.float32),
        scratch_types=[
            pltpu.VMEM((n_rows,), jnp.float32),
            pltpu.VMEM((jpw, n - (N_SPLIT // 128) * 128), jnp.float32),
            pltpu.VMEM((jpw, n_rows), jnp.float32),
        ] + [pltpu.VMEM((g, jpw, h_dim), jnp.float32)] * SC_NBUF + [
            pltpu.VMEM((jpw, h_dim), jnp.float32),
        ] + [pltpu.SemaphoreType.DMA] * SC_NBUF,
    )
    def sc_kernel(hs_hbm, skip_t_hbm, ip_hbm, out_h_hbm,
                  ip_v, skip_v, wg_v, *rest):
        bufs = rest[:SC_NBUF]
        acc_v = rest[SC_NBUF]
        sems = rest[SC_NBUF + 1:]
        wid = lax.axis_index("s") * nc + lax.axis_index("c")
        j0 = wid * jpw

        base = (N_SPLIT // 128) * 128
        off = N_SPLIT - base
        pltpu.sync_copy(ip_hbm.at[pl.ds(N_SPLIT, n_rows)], ip_v)
        pltpu.sync_copy(
            skip_t_hbm.at[pl.ds(j0, jpw), pl.ds(base, n - base)], skip_v)

        for jj in range(jpw):
            for ic in range(n_rows // nl):
                i_lane = N_SPLIT + ic * nl + lax.iota(jnp.int32, nl)
                wv = (ip_v[pl.ds(ic * nl, nl)]
                      * skip_v[jj, pl.ds(off + ic * nl, nl)])
                wv = jnp.where(i_lane == j0 + jj, 0.0, wv)
                wg_v[jj, pl.ds(ic * nl, nl)] = wv

        for jj in range(jpw):
            for hh in range(nh):
                acc_v[jj, pl.ds(hh * nl, nl)] = jnp.zeros((nl,), jnp.float32)

        def dma(grp, buf, sem):
            src = hs_hbm.at[pl.ds(N_SPLIT + grp * g, g), pl.ds(j0, jpw), :]
            return pltpu.make_async_copy(src, buf, sem)

        gdn = lax.GatherDimensionNumbers(
            offset_dims=(), collapsed_slice_dims=(0,), start_index_map=(0,))

        def splat_lane(vec, lane):
            idx = jnp.full((nl, 1), lane, jnp.int32)
            return lax.gather(
                vec, idx, gdn, (1,),
                mode=lax.GatherScatterMode.PROMISE_IN_BOUNDS)

        def accum_group(chunk_base, lane0, buf):
            for jj in range(jpw):
                w_chunk = wg_v[jj, pl.ds(chunk_base, nl)]
                rc = [jnp.zeros((nl,), jnp.float32) for _ in range(nh)]
                for gg in range(g):
                    wb = splat_lane(w_chunk, lane0 + gg)
                    for hh in range(nh):
                        rc[hh] = rc[hh] + wb * buf[gg, jj, pl.ds(hh * nl, nl)]
                for hh in range(nh):
                    plsc.addupdate(acc_v.at[jj, pl.ds(hh * nl, nl)], rc[hh])

        for b in range(SC_NBUF - 1):
            dma(b, bufs[b], sems[b]).start()

        niter = ngrp // SC_NBUF

        def outer(t, carry):
            gb = SC_NBUF * t
            chunk_base = (gb * g // nl) * nl
            for b in range(SC_NBUF):
                dma(gb + b, bufs[b], sems[b]).wait()
                nxt = gb + b + SC_NBUF - 1
                if b == 0:
                    dma(nxt, bufs[(b - 1) % SC_NBUF],
                        sems[(b - 1) % SC_NBUF]).start()
                else:
                    @pl.when(t < niter - 1)
                    def _prefetch(nxt=nxt, b=b):
                        dma(nxt, bufs[b - 1], sems[b - 1]).start()
                accum_group(chunk_base + (b * g // nl) * nl,
                            (b * g) % nl, bufs[b])
            return carry

        lax.fori_loop(0, niter, outer, 0)

        pltpu.sync_copy(acc_v, out_h_hbm.at[pl.ds(j0, jpw), :])

    return sc_kernel


def _combine_body(tcip_ref, tch_ref, sch_ref, out_ip_ref, out_h_ref):
    new_ip = tcip_ref[...]
    out_ip_ref[...] = new_ip
    out_h_ref[...] = (tch_ref[...] + sch_ref[...]) / (new_ip + 1e-7)


@jax.jit
def kernel(instruction_pointer, hidden_state_proposals,
           hidden_state_skip_proposals, skip_decisions, branch_decisions,
           true_indexes, false_indexes):
    n = instruction_pointer.shape[0]
    h_dim = hidden_state_proposals.shape[1]

    ip2 = instruction_pointer.reshape(n, 1)
    ti2 = true_indexes.reshape(1, n)
    fi2 = false_indexes.reshape(1, n)

    sc_h = _make_sc_partial(n, h_dim)(
        hidden_state_skip_proposals, skip_decisions.T, instruction_pointer)

    tc_ip, tc_h = _tc_partial(ip2, hidden_state_proposals,
                              hidden_state_skip_proposals, skip_decisions,
                              branch_decisions, ti2, fi2)

    out_ip, out_h = pl.pallas_call(
        _combine_body,
        out_shape=[
            jax.ShapeDtypeStruct((n, 1), jnp.float32),
            jax.ShapeDtypeStruct((n, h_dim), jnp.float32),
        ],
    )(tc_ip, tc_h, sc_h)

    return out_ip.reshape(n), out_h


# speedup vs baseline: 1.2753x; 1.2753x over previous
import functools

import jax
import jax.numpy as jnp
from jax.experimental import pallas as pl
from jax.experimental.pallas import tpu as pltpu


def _body(ip_ref, h_ref, hs_ref, skip_ref, br_ref, ti_ref, fi_ref,
          out_ip_ref, out_h_ref, acc_h_ref, *, block_i: int):
    k = pl.program_id(0)
    nk = pl.num_programs(0)
    bi = block_i
    n = skip_ref.shape[1]

    @pl.when(k == 0)
    def _segment_part():
        acc_h_ref[...] = jnp.zeros_like(acc_h_ref)

        rows = jax.lax.broadcasted_iota(jnp.int32, (n, n), 0)
        cols = jax.lax.broadcasted_iota(jnp.int32, (n, n), 1)
        skip_full = skip_ref[...]
        no_skip = jnp.sum(jnp.where(rows == cols, skip_full, 0.0),
                          axis=1, keepdims=True)
        ip = ip_ref[...]
        pbt = ip * no_skip * br_ref[:, 0:1]
        pbf = ip * no_skip * br_ref[:, 1:2]

        ot = (ti_ref[...] == rows).astype(jnp.float32)
        of = (fi_ref[...] == rows).astype(jnp.float32)

        dn = (((1,), (0,)), ((), ()))
        hp = jax.lax.Precision.HIGHEST
        ip_t = jax.lax.dot_general(ot, pbt, dn, precision=hp,
                                   preferred_element_type=jnp.float32)
        ip_f = jax.lax.dot_general(of, pbf, dn, precision=hp,
                                   preferred_element_type=jnp.float32)
        out_ip_ref[...] = ip_t + ip_f

        h = h_ref[...]
        th = jax.lax.dot_general(ot, h * pbt, dn, precision=hp,
                                 preferred_element_type=jnp.float32)
        fh = jax.lax.dot_general(of, h * pbf, dn, precision=hp,
                                 preferred_element_type=jnp.float32)
        out_h_ref[...] = th + fh

    row_ids = k * bi + jax.lax.broadcasted_iota(jnp.int32, (bi, n), 0)
    col_ids = jax.lax.broadcasted_iota(jnp.int32, (bi, n), 1)
    skip_blk = skip_ref[pl.ds(k * bi, bi), :]
    ip_blk = ip_ref[pl.ds(k * bi, bi), :]

    w = jnp.where(row_ids != col_ids, skip_blk, 0.0) * ip_blk

    ones = jnp.ones((bi, 1), jnp.float32)
    out_ip_ref[...] += jax.lax.dot_general(
        w, ones, (((0,), (0,)), ((), ())),
        preferred_element_type=jnp.float32)

    cj = 64
    for jc in range(n // cj):
        js = jc * cj
        hs_c = hs_ref[:, js:js + cj, :]
        w_c = w[:, js:js + cj]
        acc_h_ref[js:js + cj, :] += jnp.sum(hs_c * w_c[:, :, None], axis=0)

    @pl.when(k == nk - 1)
    def _finish():
        new_ip = out_ip_ref[...]
        out_h_ref[...] = (out_h_ref[...] + acc_h_ref[...]) / (new_ip + 1e-7)


@jax.jit
def kernel(instruction_pointer, hidden_state_proposals,
           hidden_state_skip_proposals, skip_decisions, branch_decisions,
           true_indexes, false_indexes):
    n = instruction_pointer.shape[0]
    h_dim = hidden_state_proposals.shape[1]
    block_i = 64
    nk = n // block_i

    ip2 = instruction_pointer.reshape(n, 1)
    ti2 = true_indexes.reshape(1, n)
    fi2 = false_indexes.reshape(1, n)

    out_ip, out_h = pl.pallas_call(
        functools.partial(_body, block_i=block_i),
        grid=(nk,),
        in_specs=[
            pl.BlockSpec((n, 1), lambda k: (0, 0)),
            pl.BlockSpec((n, h_dim), lambda k: (0, 0)),
            pl.BlockSpec((block_i, n, h_dim), lambda k: (k, 0, 0)),
            pl.BlockSpec((n, n), lambda k: (0, 0)),
            pl.BlockSpec((n, 2), lambda k: (0, 0)),
            pl.BlockSpec((1, n), lambda k: (0, 0)),
            pl.BlockSpec((1, n), lambda k: (0, 0)),
        ],
        out_specs=[
            pl.BlockSpec((n, 1), lambda k: (0, 0)),
            pl.BlockSpec((n, h_dim), lambda k: (0, 0)),
        ],
        out_shape=[
            jax.ShapeDtypeStruct((n, 1), jnp.float32),
            jax.ShapeDtypeStruct((n, h_dim), jnp.float32),
        ],
        scratch_shapes=[
            pltpu.VMEM((n, h_dim), jnp.float32),
        ],
    )(ip2, hidden_state_proposals, hidden_state_skip_proposals,
      skip_decisions, branch_decisions, ti2, fi2)

    return out_ip.reshape(n), out_h


# speedup vs baseline: 1.2886x; 1.0105x over previous
import functools

import jax
import jax.numpy as jnp
from jax.experimental import pallas as pl
from jax.experimental.pallas import tpu as pltpu


def _body(ip_ref, h_ref, hs_ref, skip_ref, br_ref, ti_ref, fi_ref,
          out_ip_ref, out_h_ref, acc_h_ref, pbf_ref, *, block_i: int):
    k = pl.program_id(0)
    nk = pl.num_programs(0)
    bi = block_i
    n = skip_ref.shape[1]
    dn = (((1,), (0,)), ((), ()))
    hp = jax.lax.Precision.HIGHEST

    @pl.when(k == 0)
    def _segment_true():
        acc_h_ref[...] = jnp.zeros_like(acc_h_ref)

        rows = jax.lax.broadcasted_iota(jnp.int32, (n, n), 0)
        cols = jax.lax.broadcasted_iota(jnp.int32, (n, n), 1)
        skip_full = skip_ref[...]
        no_skip = jnp.sum(jnp.where(rows == cols, skip_full, 0.0),
                          axis=1, keepdims=True)
        ip = ip_ref[...]
        pbt = ip * no_skip * br_ref[:, 0:1]
        pbf_ref[...] = ip * no_skip * br_ref[:, 1:2]

        ot = (ti_ref[...] == rows).astype(jnp.float32)
        ip_t = jax.lax.dot_general(ot, pbt, dn, precision=hp,
                                   preferred_element_type=jnp.float32)
        out_ip_ref[...] = ip_t
        th = jax.lax.dot_general(ot, h_ref[...] * pbt, dn, precision=hp,
                                 preferred_element_type=jnp.float32)
        out_h_ref[...] = th

    @pl.when(k == 1)
    def _segment_false():
        rows = jax.lax.broadcasted_iota(jnp.int32, (n, n), 0)
        pbf = pbf_ref[...]
        of = (fi_ref[...] == rows).astype(jnp.float32)
        ip_f = jax.lax.dot_general(of, pbf, dn, precision=hp,
                                   preferred_element_type=jnp.float32)
        out_ip_ref[...] += ip_f
        fh = jax.lax.dot_general(of, h_ref[...] * pbf, dn, precision=hp,
                                 preferred_element_type=jnp.float32)
        out_h_ref[...] += fh

    row_ids = k * bi + jax.lax.broadcasted_iota(jnp.int32, (bi, n), 0)
    col_ids = jax.lax.broadcasted_iota(jnp.int32, (bi, n), 1)
    skip_blk = skip_ref[pl.ds(k * bi, bi), :]
    ip_blk = ip_ref[pl.ds(k * bi, bi), :]

    w = jnp.where(row_ids != col_ids, skip_blk, 0.0) * ip_blk

    ones = jnp.ones((bi, 1), jnp.float32)
    out_ip_ref[...] += jax.lax.dot_general(
        w, ones, (((0,), (0,)), ((), ())),
        preferred_element_type=jnp.float32)

    cj = 64
    for jc in range(n // cj):
        js = jc * cj
        hs_c = hs_ref[:, js:js + cj, :]
        w_c = w[:, js:js + cj]
        acc_h_ref[js:js + cj, :] += jnp.sum(hs_c * w_c[:, :, None], axis=0)

    @pl.when(k == nk - 1)
    def _finish():
        new_ip = out_ip_ref[...]
        out_h_ref[...] = (out_h_ref[...] + acc_h_ref[...]) / (new_ip + 1e-7)


@jax.jit
def kernel(instruction_pointer, hidden_state_proposals,
           hidden_state_skip_proposals, skip_decisions, branch_decisions,
           true_indexes, false_indexes):
    n = instruction_pointer.shape[0]
    h_dim = hidden_state_proposals.shape[1]
    block_i = 64
    nk = n // block_i

    ip2 = instruction_pointer.reshape(n, 1)
    ti2 = true_indexes.reshape(1, n)
    fi2 = false_indexes.reshape(1, n)

    out_ip, out_h = pl.pallas_call(
        functools.partial(_body, block_i=block_i),
        grid=(nk,),
        in_specs=[
            pl.BlockSpec((n, 1), lambda k: (0, 0)),
            pl.BlockSpec((n, h_dim), lambda k: (0, 0)),
            pl.BlockSpec((block_i, n, h_dim), lambda k: (k, 0, 0)),
            pl.BlockSpec((n, n), lambda k: (0, 0)),
            pl.BlockSpec((n, 2), lambda k: (0, 0)),
            pl.BlockSpec((1, n), lambda k: (0, 0)),
            pl.BlockSpec((1, n), lambda k: (0, 0)),
        ],
        out_specs=[
            pl.BlockSpec((n, 1), lambda k: (0, 0)),
            pl.BlockSpec((n, h_dim), lambda k: (0, 0)),
        ],
        out_shape=[
            jax.ShapeDtypeStruct((n, 1), jnp.float32),
            jax.ShapeDtypeStruct((n, h_dim), jnp.float32),
        ],
        scratch_shapes=[
            pltpu.VMEM((n, h_dim), jnp.float32),
            pltpu.VMEM((n, 1), jnp.float32),
        ],
    )(ip2, hidden_state_proposals, hidden_state_skip_proposals,
      skip_decisions, branch_decisions, ti2, fi2)

    return out_ip.reshape(n), out_h
